# baseline (speedup 1.0000x reference)
import functools

import jax
import jax.numpy as jnp
import numpy as np
from jax.experimental import pallas as pl
from jax.experimental.pallas import tpu as pltpu


def _round_up(v, m):
    return ((v + m - 1) // m) * m


def _tree_reduce(vals, op):
    vals = list(vals)
    while len(vals) > 1:
        nxt = [op(vals[i], vals[i + 1]) for i in range(0, len(vals) - 1, 2)]
        if len(vals) % 2:
            nxt.append(vals[-1])
        vals = nxt
    return vals[0]


def _sa_body(x_ref, t_ref, corr_ref, s_ref, o_ref, pad_ref, *, C, H, W, P):
    G = min(8, C)

    def body(i, carry):
        m, su = carry
        base = pl.multiple_of(i * G, G)
        cs8 = x_ref[0, pl.ds(base, G)]
        cs = [cs8[k] for k in range(G)]
        m1 = _tree_reduce(cs, jnp.maximum)
        s1 = _tree_reduce(cs, jnp.add)
        return jnp.maximum(m, m1), su + s1

    init = (jnp.full((H, W), -jnp.inf, jnp.float32),
            jnp.zeros((H, W), jnp.float32))
    mx, sm = jax.lax.fori_loop(0, C // G, body, init, unroll=2)

    c0 = jnp.mean(mx)
    c1 = jnp.mean(sm)

    pad_rows = pad_ref.shape[0]
    pad_ref[pl.ds(0, P), :] = jnp.zeros((P, 2 * W), jnp.float32)
    pad_ref[pl.ds(P + H, pad_rows - P - H), :] = (
        jnp.zeros((pad_rows - P - H, 2 * W), jnp.float32))
    pad_ref[pl.ds(P, H), 0:W] = mx - c0
    pad_ref[pl.ds(P, H), W:2 * W] = sm - c1

    acc = None
    for ki in range(2 * P + 1):
        win = pad_ref[pl.ds(ki, H), :]
        mm = jnp.dot(win, t_ref[ki], preferred_element_type=jnp.float32)
        acc = mm if acc is None else acc + mm

    z = acc + c0 * corr_ref[0] + c1 * corr_ref[1] + s_ref[0]
    o_ref[0, 0] = jax.nn.sigmoid(z).astype(o_ref.dtype)


def _build_conv_mats(w_all, wth, kk):
    p = kk // 2
    jd = np.arange(wth)[:, None]
    jj = np.arange(wth)[None, :]
    tb = np.stack([(jd - jj + p == kj).astype(np.float32)
                   for kj in range(kk)])
    mats = jnp.einsum('ikl,ljm->kijm', w_all, tb)
    return mats.reshape(kk, 2 * wth, wth)


def _spatial_attention(x, weight, bias):
    B, C, H, W = x.shape
    kk = weight.shape[2]
    p = kk // 2
    assert C % 8 == 0 and H >= kk

    scale = jnp.array([1.0, 1.0 / C], jnp.float32)
    w_all = weight[0].astype(jnp.float32) * scale[:, None, None]
    mats = _build_conv_mats(w_all, W, kk)

    hh = np.arange(H)[:, None] + np.arange(kk)[None, :] - p
    um = ((hh >= 0) & (hh < H)).astype(np.float32)
    wwv = np.arange(W)[:, None] + np.arange(kk)[None, :] - p
    vm = ((wwv >= 0) & (wwv < W)).astype(np.float32)
    corr = jnp.einsum('hk,ikl,wl->ihw', um, w_all, vm)

    bias_s = bias.reshape(-1).astype(jnp.float32)

    pad_rows = _round_up(H + 2 * p, 8)
    body = functools.partial(_sa_body, C=C, H=H, W=W, P=p)

    cost = pl.CostEstimate(
        flops=int(B * H * W * (2 * C + 4 * kk * kk + 4)),
        transcendentals=int(B * H * W),
        bytes_accessed=int(B * (C + 1) * H * W * 4 + mats.size * 4),
    )

    return pl.pallas_call(
        body,
        out_shape=jax.ShapeDtypeStruct((B, 1, H, W), x.dtype),
        grid=(B,),
        in_specs=[
            pl.BlockSpec((1, C, H, W), lambda b: (b, 0, 0, 0)),
            pl.BlockSpec((kk, 2 * W, W), lambda b: (0, 0, 0)),
            pl.BlockSpec((2, H, W), lambda b: (0, 0, 0)),
            pl.BlockSpec(memory_space=pltpu.MemorySpace.SMEM),
        ],
        out_specs=pl.BlockSpec((1, 1, H, W), lambda b: (b, 0, 0, 0)),
        scratch_shapes=[
            pltpu.VMEM((pad_rows, 2 * W), jnp.float32),
        ],
        compiler_params=pltpu.CompilerParams(
            dimension_semantics=("parallel",),
            vmem_limit_bytes=48 * 1024 * 1024),
        cost_estimate=cost,
    )(x, mats, corr, bias_s)


def kernel(x, weight, bias):
    return _spatial_attention(x, weight, bias)


# speedup vs baseline: 1.6703x; 1.6703x over previous
import functools

import jax
import jax.numpy as jnp
import numpy as np
from jax.experimental import pallas as pl
from jax.experimental.pallas import tpu as pltpu


def _round_up(v, m):
    return ((v + m - 1) // m) * m


def _tree_reduce(vals, op):
    vals = list(vals)
    while len(vals) > 1:
        nxt = [op(vals[i], vals[i + 1]) for i in range(0, len(vals) - 1, 2)]
        if len(vals) % 2:
            nxt.append(vals[-1])
        vals = nxt
    return vals[0]


def _sa_body(x_lo_ref, x_hi_ref, m_ref, corr_ref, s_ref, o_ref, pad_ref, *,
             C, HPAIR, W2):
    ch = x_lo_ref.shape[1]
    rows = 8
    cpi = min(4, ch // rows)
    n_iter = ch // (rows * cpi)
    step_c = rows * cpi

    for s in range(HPAIR):
        lane0 = s * W2

        def body(i, carry, _lane0=lane0):
            m, su = carry
            base = pl.multiple_of(i * step_c, step_c)
            cs = []
            for ref in (x_lo_ref, x_hi_ref):
                cs += [ref[0, pl.ds(base + k * rows, rows), pl.ds(_lane0, W2)]
                       for k in range(cpi)]
            m1 = _tree_reduce(cs, jnp.maximum)
            s1 = _tree_reduce(cs, jnp.add)
            return jnp.maximum(m, m1), su + s1

        init = (jnp.full((rows, W2), -jnp.inf, jnp.float32),
                jnp.zeros((rows, W2), jnp.float32))
        mx, sm = jax.lax.fori_loop(0, n_iter, body, init, unroll=2)
        pad_ref[pl.ds(2 + s, 1), 0:W2] = jnp.max(mx, axis=0, keepdims=True)
        pad_ref[pl.ds(2 + s, 1), W2:2 * W2] = jnp.sum(sm, axis=0,
                                                      keepdims=True)

    blk = pad_ref[pl.ds(2, HPAIR), :]
    c0 = jnp.mean(blk[:, 0:W2])
    c1 = jnp.mean(blk[:, W2:2 * W2])
    lane = jax.lax.broadcasted_iota(jnp.int32, (HPAIR, 2 * W2), 1)
    offs = jnp.where(lane < W2, c0, c1)
    pad_ref[0:2, :] = jnp.zeros((2, 2 * W2), jnp.float32)
    pad_ref[pl.ds(2 + HPAIR, 2), :] = jnp.zeros((2, 2 * W2), jnp.float32)
    pad_ref[pl.ds(2, HPAIR), :] = blk - offs

    acc = None
    for dlt in range(5):
        win = pad_ref[pl.ds(dlt, HPAIR), :]
        mm = jnp.dot(win, m_ref[dlt], preferred_element_type=jnp.float32)
        acc = mm if acc is None else acc + mm

    z = acc + c0 * corr_ref[0] + c1 * corr_ref[1] + s_ref[0]
    o_ref[0, 0] = jax.nn.sigmoid(z).astype(o_ref.dtype)


def _build_conv_mats(w_all, wth, kk):
    p = kk // 2
    a = np.zeros((5, 2, 2, kk), np.float32)
    for dlt in range(5):
        for pp in (0, 1):
            for q in (0, 1):
                ki = 2 * (dlt - 2) + pp - q + p
                if 0 <= ki < kk:
                    a[dlt, pp, q, ki] = 1.0
    jd = np.arange(wth)[:, None]
    jj = np.arange(wth)[None, :]
    tb = np.stack([(jd - jj + p == kj).astype(np.float32)
                   for kj in range(kk)])
    t1 = jnp.einsum('dpqk,ikl->dpqil', a, w_all)
    mats = jnp.einsum('dpqil,ljm->dipjqm', t1, tb)
    return mats.reshape(5, 4 * wth, 2 * wth)


def _spatial_attention(x, weight, bias):
    B, C, H, W = x.shape
    kk = weight.shape[2]
    p = kk // 2
    assert H % 2 == 0 and C % 16 == 0
    hpair = H // 2
    w2 = 2 * W

    x_flat = x.reshape(B, C, H * W)

    scale = jnp.array([1.0, 1.0 / C], jnp.float32)
    w_all = weight[0].astype(jnp.float32) * scale[:, None, None]
    mats = _build_conv_mats(w_all, W, kk)

    hh = np.arange(H)[:, None] + np.arange(kk)[None, :] - p
    um = ((hh >= 0) & (hh < H)).astype(np.float32)
    wwv = np.arange(W)[:, None] + np.arange(kk)[None, :] - p
    vm = ((wwv >= 0) & (wwv < W)).astype(np.float32)
    corr = jnp.einsum('hk,ikl,wl->ihw', um, w_all, vm).reshape(2, hpair, w2)

    bias_s = bias.reshape(-1).astype(jnp.float32)

    pad_rows = _round_up(hpair + 4, 8)
    body = functools.partial(_sa_body, C=C, HPAIR=hpair, W2=w2)

    cost = pl.CostEstimate(
        flops=int(B * H * W * (2 * C + 4 * kk * kk + 4)),
        transcendentals=int(B * H * W),
        bytes_accessed=int(B * (C + 1) * H * W * 4 + mats.size * 4),
    )

    out = pl.pallas_call(
        body,
        out_shape=jax.ShapeDtypeStruct((B, 1, hpair, w2), x.dtype),
        grid=(B,),
        in_specs=[
            pl.BlockSpec((1, C // 2, H * W), lambda b: (b, 0, 0)),
            pl.BlockSpec((1, C // 2, H * W), lambda b: (b, 1, 0)),
            pl.BlockSpec((5, 4 * W, w2), lambda b: (0, 0, 0)),
            pl.BlockSpec((2, hpair, w2), lambda b: (0, 0, 0)),
            pl.BlockSpec(memory_space=pltpu.MemorySpace.SMEM),
        ],
        out_specs=pl.BlockSpec((1, 1, hpair, w2), lambda b: (b, 0, 0, 0)),
        scratch_shapes=[
            pltpu.VMEM((pad_rows, 2 * w2), jnp.float32),
        ],
        compiler_params=pltpu.CompilerParams(
            dimension_semantics=("parallel",),
            vmem_limit_bytes=48 * 1024 * 1024),
        cost_estimate=cost,
    )(x_flat, x_flat, mats, corr, bias_s)

    return out.reshape(B, 1, H, W)


def kernel(x, weight, bias):
    return _spatial_attention(x, weight, bias)


# speedup vs baseline: 1.6913x; 1.0125x over previous
import functools

import jax
import jax.numpy as jnp
import numpy as np
from jax.experimental import pallas as pl
from jax.experimental.pallas import tpu as pltpu


def _round_up(v, m):
    return ((v + m - 1) // m) * m


def _tree_reduce(vals, op):
    vals = list(vals)
    while len(vals) > 1:
        nxt = [op(vals[i], vals[i + 1]) for i in range(0, len(vals) - 1, 2)]
        if len(vals) % 2:
            nxt.append(vals[-1])
        vals = nxt
    return vals[0]


def _sa_body(x0_ref, x1_ref, x2_ref, x3_ref, m_ref, corr_ref, s_ref,
             o_ref, pad_ref, *, C, HPAIR, W2):
    x_refs = (x0_ref, x1_ref, x2_ref, x3_ref)
    ch = x0_ref.shape[1]
    rows = 8
    cpi = max(1, min(2, ch // rows))
    n_iter = ch // (rows * cpi)
    step_c = rows * cpi

    for s in range(HPAIR):
        lane0 = s * W2

        def body(i, carry, _lane0=lane0):
            m, su = carry
            base = pl.multiple_of(i * step_c, step_c)
            cs = []
            for ref in x_refs:
                cs += [ref[0, pl.ds(base + k * rows, rows), pl.ds(_lane0, W2)]
                       for k in range(cpi)]
            m1 = _tree_reduce(cs, jnp.maximum)
            s1 = _tree_reduce(cs, jnp.add)
            return jnp.maximum(m, m1), su + s1

        init = (jnp.full((rows, W2), -jnp.inf, jnp.float32),
                jnp.zeros((rows, W2), jnp.float32))
        mx, sm = jax.lax.fori_loop(0, n_iter, body, init, unroll=2)
        pad_ref[pl.ds(2 + s, 1), 0:W2] = jnp.max(mx, axis=0, keepdims=True)
        pad_ref[pl.ds(2 + s, 1), W2:2 * W2] = jnp.sum(sm, axis=0,
                                                      keepdims=True)

    blk = pad_ref[pl.ds(2, HPAIR), :]
    c0 = jnp.mean(blk[:, 0:W2])
    c1 = jnp.mean(blk[:, W2:2 * W2])
    lane = jax.lax.broadcasted_iota(jnp.int32, (HPAIR, 2 * W2), 1)
    offs = jnp.where(lane < W2, c0, c1)
    pad_ref[0:2, :] = jnp.zeros((2, 2 * W2), jnp.float32)
    pad_ref[pl.ds(2 + HPAIR, 2), :] = jnp.zeros((2, 2 * W2), jnp.float32)
    pad_ref[pl.ds(2, HPAIR), :] = blk - offs

    acc = None
    for dlt in range(5):
        win = pad_ref[pl.ds(dlt, HPAIR), :]
        mm = jnp.dot(win, m_ref[dlt], preferred_element_type=jnp.float32)
        acc = mm if acc is None else acc + mm

    z = acc + c0 * corr_ref[0] + c1 * corr_ref[1] + s_ref[0]
    zs = jax.nn.sigmoid(z).astype(o_ref.dtype)
    w = W2 // 2
    for r in range(HPAIR):
        o_ref[0, 0, pl.ds(2 * r, 1), :] = zs[r:r + 1, 0:w]
        o_ref[0, 0, pl.ds(2 * r + 1, 1), :] = zs[r:r + 1, w:W2]


def _build_conv_mats(w_all, wth, kk, w_all_scale2):
    p = kk // 2
    plane_scale = np.array([1.0, w_all_scale2], np.float32)
    a = np.zeros((5, 2, 2, 2, kk), np.float32)
    for dlt in range(5):
        for pp in (0, 1):
            for q in (0, 1):
                ki = 2 * (dlt - 2) + pp - q + p
                if 0 <= ki < kk:
                    a[dlt, pp, q, :, ki] = plane_scale
    jd = np.arange(wth)[:, None]
    jj = np.arange(wth)[None, :]
    tb = np.stack([(jd - jj + p == kj).astype(np.float32)
                   for kj in range(kk)])
    t1 = jnp.einsum('dpqik,ikl->dpqil', a, w_all)
    mats = jnp.einsum('dpqil,ljm->dipjqm', t1, tb)
    return mats.reshape(5, 4 * wth, 2 * wth)


def _spatial_attention(x, weight, bias):
    B, C, H, W = x.shape
    kk = weight.shape[2]
    p = kk // 2
    assert H % 2 == 0 and C % 32 == 0
    hpair = H // 2
    w2 = 2 * W

    x_flat = x.reshape(B, C, H * W)

    w_all = weight[0].astype(jnp.float32)
    mats = _build_conv_mats(w_all, W, kk, 1.0 / C)

    hh = np.arange(H)[:, None] + np.arange(kk)[None, :] - p
    um = ((hh >= 0) & (hh < H)).astype(np.float32)
    um2 = um[None] * np.array([1.0, 1.0 / C], np.float32)[:, None, None]
    wwv = np.arange(W)[:, None] + np.arange(kk)[None, :] - p
    vm = ((wwv >= 0) & (wwv < W)).astype(np.float32)
    corr = jnp.einsum('ihk,ikl,wl->ihw', um2, w_all, vm).reshape(2, hpair, w2)

    bias_s = bias.reshape(-1).astype(jnp.float32)

    pad_rows = _round_up(hpair + 4, 8)
    body = functools.partial(_sa_body, C=C, HPAIR=hpair, W2=w2)

    cost = pl.CostEstimate(
        flops=int(B * H * W * (2 * C + 4 * kk * kk + 4)),
        transcendentals=int(B * H * W),
        bytes_accessed=int(B * (C + 1) * H * W * 4 + mats.size * 4),
    )

    return pl.pallas_call(
        body,
        out_shape=jax.ShapeDtypeStruct((B, 1, H, W), x.dtype),
        grid=(B,),
        in_specs=[
            pl.BlockSpec((1, C // 4, H * W), lambda b: (b, 0, 0)),
            pl.BlockSpec((1, C // 4, H * W), lambda b: (b, 1, 0)),
            pl.BlockSpec((1, C // 4, H * W), lambda b: (b, 2, 0)),
            pl.BlockSpec((1, C // 4, H * W), lambda b: (b, 3, 0)),
            pl.BlockSpec((5, 4 * W, w2), lambda b: (0, 0, 0)),
            pl.BlockSpec((2, hpair, w2), lambda b: (0, 0, 0)),
            pl.BlockSpec(memory_space=pltpu.MemorySpace.SMEM),
        ],
        out_specs=pl.BlockSpec((1, 1, H, W), lambda b: (b, 0, 0, 0)),
        scratch_shapes=[
            pltpu.VMEM((pad_rows, 2 * w2), jnp.float32),
        ],
        compiler_params=pltpu.CompilerParams(
            dimension_semantics=("parallel",),
            vmem_limit_bytes=48 * 1024 * 1024),
        cost_estimate=cost,
    )(x_flat, x_flat, x_flat, x_flat, mats, corr, bias_s)


def kernel(x, weight, bias):
    return _spatial_attention(x, weight, bias)


# speedup vs baseline: 1.7649x; 1.0435x over previous
import functools

import jax
import jax.numpy as jnp
import numpy as np
from jax.experimental import pallas as pl
from jax.experimental.pallas import tpu as pltpu


def _round_up(v, m):
    return ((v + m - 1) // m) * m


def _tree_reduce(vals, op):
    vals = list(vals)
    while len(vals) > 1:
        nxt = [op(vals[i], vals[i + 1]) for i in range(0, len(vals) - 1, 2)]
        if len(vals) % 2:
            nxt.append(vals[-1])
        vals = nxt
    return vals[0]


def _sa_body(x0_ref, x1_ref, x2_ref, x3_ref, m_ref, corr_ref, s_ref,
             o_ref, pad_ref, *, C, HPAIR, W2):
    x_refs = (x0_ref, x1_ref, x2_ref, x3_ref)
    bb_blk = x0_ref.shape[0]
    ch = x0_ref.shape[1]
    rows = 8
    cpi = max(1, min(2, ch // rows))
    n_iter = ch // (rows * cpi)
    step_c = rows * cpi

    for bb in range(bb_blk):
        for s in range(HPAIR):
            lane0 = s * W2

            def body(i, carry, _lane0=lane0, _bb=bb):
                m, su = carry
                base = pl.multiple_of(i * step_c, step_c)
                cs = []
                for ref in x_refs:
                    cs += [ref[_bb, pl.ds(base + k * rows, rows),
                               pl.ds(_lane0, W2)] for k in range(cpi)]
                m1 = _tree_reduce(cs, jnp.maximum)
                s1 = _tree_reduce(cs, jnp.add)
                return jnp.maximum(m, m1), su + s1

            init = (jnp.full((rows, W2), -jnp.inf, jnp.float32),
                    jnp.zeros((rows, W2), jnp.float32))
            mx, sm = jax.lax.fori_loop(0, n_iter, body, init, unroll=2)
            pad_ref[pl.ds(2 + s, 1), 0:W2] = jnp.max(mx, axis=0,
                                                     keepdims=True)
            pad_ref[pl.ds(2 + s, 1), W2:2 * W2] = jnp.sum(sm, axis=0,
                                                          keepdims=True)

        blk = pad_ref[pl.ds(2, HPAIR), :]
        c0 = jnp.mean(blk[:, 0:W2])
        c1 = jnp.mean(blk[:, W2:2 * W2])
        lane = jax.lax.broadcasted_iota(jnp.int32, (HPAIR, 2 * W2), 1)
        offs = jnp.where(lane < W2, c0, c1)
        pad_ref[0:2, :] = jnp.zeros((2, 2 * W2), jnp.float32)
        pad_ref[pl.ds(2 + HPAIR, 2), :] = jnp.zeros((2, 2 * W2), jnp.float32)
        pad_ref[pl.ds(2, HPAIR), :] = blk - offs

        acc = None
        for dlt in range(5):
            win = pad_ref[pl.ds(dlt, HPAIR), :]
            mm = jnp.dot(win, m_ref[dlt], preferred_element_type=jnp.float32)
            acc = mm if acc is None else acc + mm

        z = acc + c0 * corr_ref[0] + c1 * corr_ref[1] + s_ref[0]
        zs = jax.nn.sigmoid(z).astype(o_ref.dtype)
        w = W2 // 2
        for r in range(HPAIR):
            o_ref[bb, 0, pl.ds(2 * r, 1), :] = zs[r:r + 1, 0:w]
            o_ref[bb, 0, pl.ds(2 * r + 1, 1), :] = zs[r:r + 1, w:W2]


def _build_conv_mats(w_all, wth, kk, w_all_scale2):
    p = kk // 2
    plane_scale = np.array([1.0, w_all_scale2], np.float32)
    a = np.zeros((5, 2, 2, 2, kk), np.float32)
    for dlt in range(5):
        for pp in (0, 1):
            for q in (0, 1):
                ki = 2 * (dlt - 2) + pp - q + p
                if 0 <= ki < kk:
                    a[dlt, pp, q, :, ki] = plane_scale
    jd = np.arange(wth)[:, None]
    jj = np.arange(wth)[None, :]
    tb = np.stack([(jd - jj + p == kj).astype(np.float32)
                   for kj in range(kk)])
    t1 = jnp.einsum('dpqik,ikl->dpqil', a, w_all)
    mats = jnp.einsum('dpqil,ljm->dipjqm', t1, tb)
    return mats.reshape(5, 4 * wth, 2 * wth)


def _spatial_attention(x, weight, bias):
    B, C, H, W = x.shape
    kk = weight.shape[2]
    p = kk // 2
    assert H % 2 == 0 and C % 32 == 0
    hpair = H // 2
    w2 = 2 * W

    x_flat = x.reshape(B, C, H * W)

    w_all = weight[0].astype(jnp.float32)
    mats = _build_conv_mats(w_all, W, kk, 1.0 / C)

    hh = np.arange(H)[:, None] + np.arange(kk)[None, :] - p
    um = ((hh >= 0) & (hh < H)).astype(np.float32)
    um2 = um[None] * np.array([1.0, 1.0 / C], np.float32)[:, None, None]
    wwv = np.arange(W)[:, None] + np.arange(kk)[None, :] - p
    vm = ((wwv >= 0) & (wwv < W)).astype(np.float32)
    corr = jnp.einsum('ihk,ikl,wl->ihw', um2, w_all, vm).reshape(2, hpair, w2)

    bias_s = bias.reshape(-1).astype(jnp.float32)

    pad_rows = _round_up(hpair + 4, 8)
    body = functools.partial(_sa_body, C=C, HPAIR=hpair, W2=w2)

    cost = pl.CostEstimate(
        flops=int(B * H * W * (2 * C + 4 * kk * kk + 4)),
        transcendentals=int(B * H * W),
        bytes_accessed=int(B * (C + 1) * H * W * 4 + mats.size * 4),
    )

    bblk = 4 if B % 4 == 0 else 1
    return pl.pallas_call(
        body,
        out_shape=jax.ShapeDtypeStruct((B, 1, H, W), x.dtype),
        grid=(B // bblk,),
        in_specs=[
            pl.BlockSpec((bblk, C // 4, H * W), lambda b: (b, 0, 0)),
            pl.BlockSpec((bblk, C // 4, H * W), lambda b: (b, 1, 0)),
            pl.BlockSpec((bblk, C // 4, H * W), lambda b: (b, 2, 0)),
            pl.BlockSpec((bblk, C // 4, H * W), lambda b: (b, 3, 0)),
            pl.BlockSpec((5, 4 * W, w2), lambda b: (0, 0, 0)),
            pl.BlockSpec((2, hpair, w2), lambda b: (0, 0, 0)),
            pl.BlockSpec(memory_space=pltpu.MemorySpace.SMEM),
        ],
        out_specs=pl.BlockSpec((bblk, 1, H, W), lambda b: (b, 0, 0, 0)),
        scratch_shapes=[
            pltpu.VMEM((pad_rows, 2 * w2), jnp.float32),
        ],
        compiler_params=pltpu.CompilerParams(
            dimension_semantics=("parallel",),
            vmem_limit_bytes=48 * 1024 * 1024),
        cost_estimate=cost,
    )(x_flat, x_flat, x_flat, x_flat, mats, corr, bias_s)


def kernel(x, weight, bias):
    return _spatial_attention(x, weight, bias)


# speedup vs baseline: 1.8664x; 1.0575x over previous
import functools

import jax
import jax.numpy as jnp
import numpy as np
from jax.experimental import pallas as pl
from jax.experimental.pallas import tpu as pltpu


def _round_up(v, m):
    return ((v + m - 1) // m) * m


def _tree_reduce(vals, op):
    vals = list(vals)
    while len(vals) > 1:
        nxt = [op(vals[i], vals[i + 1]) for i in range(0, len(vals) - 1, 2)]
        if len(vals) % 2:
            nxt.append(vals[-1])
        vals = nxt
    return vals[0]


def _sa_body(x0_ref, x1_ref, x2_ref, x3_ref, x4_ref, x5_ref, x6_ref,
             x7_ref, m_ref, corr_ref, s_ref, o_ref, pad_ref, *,
             C, HPAIR, W2):
    x_refs = (x0_ref, x1_ref, x2_ref, x3_ref, x4_ref, x5_ref, x6_ref,
              x7_ref)
    bb_blk = x0_ref.shape[0]
    ch = x0_ref.shape[1]
    rows = 8
    cpi = max(1, min(2, ch // rows))
    n_iter = ch // (rows * cpi)
    step_c = rows * cpi

    for bb in range(bb_blk):
        for s in range(HPAIR):
            lane0 = s * W2

            def body(i, carry, _lane0=lane0, _bb=bb):
                m, su = carry
                base = pl.multiple_of(i * step_c, step_c)
                cs = []
                for ref in x_refs:
                    cs += [ref[_bb, pl.ds(base + k * rows, rows),
                               pl.ds(_lane0, W2)] for k in range(cpi)]
                m1 = _tree_reduce(cs, jnp.maximum)
                s1 = _tree_reduce(cs, jnp.add)
                return jnp.maximum(m, m1), su + s1

            init = (jnp.full((rows, W2), -jnp.inf, jnp.float32),
                    jnp.zeros((rows, W2), jnp.float32))
            mx, sm = jax.lax.fori_loop(0, n_iter, body, init, unroll=2)
            pad_ref[pl.ds(2 + s, 1), 0:W2] = jnp.max(mx, axis=0,
                                                     keepdims=True)
            pad_ref[pl.ds(2 + s, 1), W2:2 * W2] = jnp.sum(sm, axis=0,
                                                          keepdims=True)

        blk = pad_ref[pl.ds(2, HPAIR), :]
        c0 = jnp.mean(blk[:, 0:W2])
        c1 = jnp.mean(blk[:, W2:2 * W2])
        lane = jax.lax.broadcasted_iota(jnp.int32, (HPAIR, 2 * W2), 1)
        offs = jnp.where(lane < W2, c0, c1)
        pad_ref[0:2, :] = jnp.zeros((2, 2 * W2), jnp.float32)
        pad_ref[pl.ds(2 + HPAIR, 2), :] = jnp.zeros((2, 2 * W2), jnp.float32)
        pad_ref[pl.ds(2, HPAIR), :] = blk - offs

        acc = None
        for dlt in range(5):
            win = pad_ref[pl.ds(dlt, HPAIR), :]
            mm = jnp.dot(win, m_ref[dlt], preferred_element_type=jnp.float32)
            acc = mm if acc is None else acc + mm

        z = acc + c0 * corr_ref[0] + c1 * corr_ref[1] + s_ref[0]
        zs = jax.nn.sigmoid(z).astype(o_ref.dtype)
        w = W2 // 2
        for r in range(HPAIR):
            o_ref[bb, 0, pl.ds(2 * r, 1), :] = zs[r:r + 1, 0:w]
            o_ref[bb, 0, pl.ds(2 * r + 1, 1), :] = zs[r:r + 1, w:W2]


def _build_conv_mats(w_all, wth, kk, w_all_scale2):
    p = kk // 2
    plane_scale = np.array([1.0, w_all_scale2], np.float32)
    a = np.zeros((5, 2, 2, 2, kk), np.float32)
    for dlt in range(5):
        for pp in (0, 1):
            for q in (0, 1):
                ki = 2 * (dlt - 2) + pp - q + p
                if 0 <= ki < kk:
                    a[dlt, pp, q, :, ki] = plane_scale
    jd = np.arange(wth)[:, None]
    jj = np.arange(wth)[None, :]
    tb = np.stack([(jd - jj + p == kj).astype(np.float32)
                   for kj in range(kk)])
    t1 = jnp.einsum('dpqik,ikl->dpqil', a, w_all)
    mats = jnp.einsum('dpqil,ljm->dipjqm', t1, tb)
    return mats.reshape(5, 4 * wth, 2 * wth)


def _spatial_attention(x, weight, bias):
    B, C, H, W = x.shape
    kk = weight.shape[2]
    p = kk // 2
    assert H % 2 == 0 and C % 64 == 0
    hpair = H // 2
    w2 = 2 * W

    x_flat = x.reshape(B, C, H * W)

    w_all = weight[0].astype(jnp.float32)
    mats = _build_conv_mats(w_all, W, kk, 1.0 / C)

    hh = np.arange(H)[:, None] + np.arange(kk)[None, :] - p
    um = ((hh >= 0) & (hh < H)).astype(np.float32)
    um2 = um[None] * np.array([1.0, 1.0 / C], np.float32)[:, None, None]
    wwv = np.arange(W)[:, None] + np.arange(kk)[None, :] - p
    vm = ((wwv >= 0) & (wwv < W)).astype(np.float32)
    corr = jnp.einsum('ihk,ikl,wl->ihw', um2, w_all, vm).reshape(2, hpair, w2)

    bias_s = bias.reshape(-1).astype(jnp.float32)

    pad_rows = _round_up(hpair + 4, 8)
    body = functools.partial(_sa_body, C=C, HPAIR=hpair, W2=w2)

    cost = pl.CostEstimate(
        flops=int(B * H * W * (2 * C + 4 * kk * kk + 4)),
        transcendentals=int(B * H * W),
        bytes_accessed=int(B * (C + 1) * H * W * 4 + mats.size * 4),
    )

    bblk = 4 if B % 4 == 0 else 1
    return pl.pallas_call(
        body,
        out_shape=jax.ShapeDtypeStruct((B, 1, H, W), x.dtype),
        grid=(B // bblk,),
        in_specs=[
            pl.BlockSpec((bblk, C // 8, H * W), lambda b: (b, 0, 0)),
            pl.BlockSpec((bblk, C // 8, H * W), lambda b: (b, 1, 0)),
            pl.BlockSpec((bblk, C // 8, H * W), lambda b: (b, 2, 0)),
            pl.BlockSpec((bblk, C // 8, H * W), lambda b: (b, 3, 0)),
            pl.BlockSpec((bblk, C // 8, H * W), lambda b: (b, 4, 0)),
            pl.BlockSpec((bblk, C // 8, H * W), lambda b: (b, 5, 0)),
            pl.BlockSpec((bblk, C // 8, H * W), lambda b: (b, 6, 0)),
            pl.BlockSpec((bblk, C // 8, H * W), lambda b: (b, 7, 0)),
            pl.BlockSpec((5, 4 * W, w2), lambda b: (0, 0, 0)),
            pl.BlockSpec((2, hpair, w2), lambda b: (0, 0, 0)),
            pl.BlockSpec(memory_space=pltpu.MemorySpace.SMEM),
        ],
        out_specs=pl.BlockSpec((bblk, 1, H, W), lambda b: (b, 0, 0, 0)),
        scratch_shapes=[
            pltpu.VMEM((pad_rows, 2 * w2), jnp.float32),
        ],
        compiler_params=pltpu.CompilerParams(
            dimension_semantics=("parallel",),
            vmem_limit_bytes=48 * 1024 * 1024),
        cost_estimate=cost,
    )(x_flat, x_flat, x_flat, x_flat, x_flat, x_flat, x_flat, x_flat,
      mats, corr, bias_s)


def kernel(x, weight, bias):
    return _spatial_attention(x, weight, bias)
